# baseline (speedup 1.0000x reference)
import math

import jax
import jax.numpy as jnp
from jax.experimental import pallas as pl
from jax.experimental.pallas import tpu as pltpu

N_ROWS = 200000
BL = 32768
NBL = -(-N_ROWS // BL)
IN_CH = 32
HID = 64
OUT = 4
MROW = 8

_CONTRACT_0_0 = (((0,), (0,)), ((), ()))
_CONTRACT_1_1 = (((1,), (1,)), ((), ()))


def _fused_kernel(x_ref, w1t_ref, gamma_ref, beta_ref, w2t_ref, b2_ref,
                  out_ref, g_ref, m_ref, w1s_ref, sh_ref, b2c_ref, w2h_ref,
                  xc_ref):
    p = pl.program_id(0)
    i = pl.program_id(1)

    @pl.when(p == 0)
    def _stats():
        x = x_ref[...]
        lane = jax.lax.broadcasted_iota(jnp.int32, (IN_CH, BL), 1)
        x = jnp.where(lane < (N_ROWS - i * BL), x, 0.0)
        xb = x.astype(jnp.bfloat16)
        xc_ref[:, pl.ds(i * BL, BL)] = xb
        g = jax.lax.dot_general(xb, xb, _CONTRACT_1_1,
                                preferred_element_type=jnp.float32)
        ones = jnp.ones((MROW, BL), dtype=jnp.bfloat16)
        m = jax.lax.dot_general(ones, xb, _CONTRACT_1_1,
                                preferred_element_type=jnp.float32)

        @pl.when(i == 0)
        def _init():
            g_ref[...] = g
            m_ref[...] = m

        @pl.when(i != 0)
        def _acc():
            g_ref[...] += g
            m_ref[...] += m

    @pl.when((p == 1) & (i == 0))
    def _finalize():
        wt = w1t_ref[...]
        mu = m_ref[0:1, :] / N_ROWS
        mean = jnp.dot(mu, wt,
                       preferred_element_type=jnp.float32)
        t = jnp.dot(g_ref[...] / N_ROWS, wt,
                    preferred_element_type=jnp.float32)
        ex2 = jnp.sum(t * wt, axis=0, keepdims=True)
        var = ex2 - mean * mean
        isq2 = 1.0 / math.sqrt(2.0)
        scale = gamma_ref[...] * jax.lax.rsqrt(var + 1e-5)
        shift = beta_ref[...] - mean * scale
        w1s_ref[...] = (wt * (scale * isq2)).astype(jnp.bfloat16)
        sh_ref[...] = (shift * isq2).T.astype(jnp.bfloat16)
        b2c_ref[...] = b2_ref[...].T
        w2h_ref[...] = (w2t_ref[...] * isq2).astype(jnp.bfloat16)

    @pl.when(p == 1)
    def _apply():
        xb = xc_ref[:, pl.ds(i * BL, BL)]
        h = jax.lax.dot_general(w1s_ref[...], xb, _CONTRACT_0_0,
                                preferred_element_type=jnp.float32)
        hn = (h.astype(jnp.bfloat16)
              + jnp.broadcast_to(sh_ref[...], (HID, BL)))
        e = jax.lax.erf(hn)
        gl2 = hn + hn * e
        o = jnp.dot(w2h_ref[...], gl2,
                    preferred_element_type=jnp.float32)
        ob = o + jnp.broadcast_to(b2c_ref[...], (OUT, BL))
        out_ref[...] = jax.nn.sigmoid(ob)


def kernel(feat, W1, b1, gamma, beta, W2, b2):
    del b1
    outT = pl.pallas_call(
        _fused_kernel,
        grid=(2, NBL),
        in_specs=[
            pl.BlockSpec((IN_CH, BL), lambda p, i: (0, i * (1 - p))),
            pl.BlockSpec((IN_CH, HID), lambda p, i: (0, 0)),
            pl.BlockSpec((1, HID), lambda p, i: (0, 0)),
            pl.BlockSpec((1, HID), lambda p, i: (0, 0)),
            pl.BlockSpec((OUT, HID), lambda p, i: (0, 0)),
            pl.BlockSpec((1, OUT), lambda p, i: (0, 0)),
        ],
        out_specs=pl.BlockSpec((OUT, BL), lambda p, i: (0, i)),
        out_shape=jax.ShapeDtypeStruct((OUT, N_ROWS), jnp.float32),
        scratch_shapes=[
            pltpu.VMEM((IN_CH, IN_CH), jnp.float32),
            pltpu.VMEM((MROW, IN_CH), jnp.float32),
            pltpu.VMEM((IN_CH, HID), jnp.bfloat16),
            pltpu.VMEM((HID, 1), jnp.bfloat16),
            pltpu.VMEM((OUT, 1), jnp.float32),
            pltpu.VMEM((OUT, HID), jnp.bfloat16),
            pltpu.VMEM((IN_CH, NBL * BL), jnp.bfloat16),
        ],
        compiler_params=pltpu.CompilerParams(
            dimension_semantics=("arbitrary", "arbitrary")),
    )(feat.T, W1.T, gamma[None, :], beta[None, :], W2, b2[None, :])

    return outT.T


# speedup vs baseline: 1.0911x; 1.0911x over previous
import math

import jax
import jax.numpy as jnp
from jax.experimental import pallas as pl
from jax.experimental.pallas import tpu as pltpu

N_ROWS = 200000
BL = 32768
NBL = -(-N_ROWS // BL)
IN_CH = 32
HID = 64
OUT = 4
MROW = 8

_CONTRACT_0_0 = (((0,), (0,)), ((), ()))
_CONTRACT_1_1 = (((1,), (1,)), ((), ()))


def _fused_kernel(x_ref, w1t_ref, gamma_ref, beta_ref, w2t_ref, b2_ref,
                  out_ref, g_ref, m_ref, w1s_ref, sh_ref, b2c_ref, w2h_ref,
                  xc_ref):
    p = pl.program_id(0)
    i = pl.program_id(1)

    @pl.when(p == 0)
    def _stats():
        x = x_ref[...]
        lane = jax.lax.broadcasted_iota(jnp.int32, (IN_CH, BL), 1)
        x = jnp.where(lane < (N_ROWS - i * BL), x, 0.0)
        xb = x.astype(jnp.bfloat16)
        xc_ref[:, pl.ds(i * BL, BL)] = xb
        g = jax.lax.dot_general(xb, xb, _CONTRACT_1_1,
                                preferred_element_type=jnp.float32)
        ones = jnp.ones((MROW, BL), dtype=jnp.bfloat16)
        m = jax.lax.dot_general(ones, xb, _CONTRACT_1_1,
                                preferred_element_type=jnp.float32)

        @pl.when(i == 0)
        def _init():
            g_ref[...] = g
            m_ref[...] = m

        @pl.when(i != 0)
        def _acc():
            g_ref[...] += g
            m_ref[...] += m

    @pl.when((p == 1) & (i == 0))
    def _finalize():
        wt = w1t_ref[...]
        mu = m_ref[0:1, :] / N_ROWS
        mean = jnp.dot(mu, wt,
                       preferred_element_type=jnp.float32)
        t = jnp.dot(g_ref[...] / N_ROWS, wt,
                    preferred_element_type=jnp.float32)
        ex2 = jnp.sum(t * wt, axis=0, keepdims=True)
        var = ex2 - mean * mean
        scale = gamma_ref[...] * jax.lax.rsqrt(var + 1e-5)
        shift = beta_ref[...] - mean * scale
        w1s_ref[...] = (wt * scale).astype(jnp.bfloat16)
        sh_ref[...] = shift.T
        b2c_ref[...] = b2_ref[...].T
        w2h_ref[...] = (w2t_ref[...] * 0.5).astype(jnp.bfloat16)

    @pl.when(p == 1)
    def _apply():
        xb = xc_ref[:, pl.ds(i * BL, BL)]
        h = jax.lax.dot_general(w1s_ref[...], xb, _CONTRACT_0_0,
                                preferred_element_type=jnp.float32)
        hn = (h + jnp.broadcast_to(sh_ref[...], (HID, BL))
              ).astype(jnp.bfloat16)
        e = jax.lax.erf(hn * jnp.bfloat16(1.0 / math.sqrt(2.0)))
        gl2 = hn + hn * e
        o = jnp.dot(w2h_ref[...], gl2,
                    preferred_element_type=jnp.float32)
        ob = o + jnp.broadcast_to(b2c_ref[...], (OUT, BL))
        out_ref[...] = jax.nn.sigmoid(ob)


def kernel(feat, W1, b1, gamma, beta, W2, b2):
    del b1
    outT = pl.pallas_call(
        _fused_kernel,
        grid=(2, NBL),
        in_specs=[
            pl.BlockSpec((IN_CH, BL),
                         lambda p, i: (0, i * (1 - p) + (NBL - 1) * p)),
            pl.BlockSpec((IN_CH, HID), lambda p, i: (0, 0)),
            pl.BlockSpec((1, HID), lambda p, i: (0, 0)),
            pl.BlockSpec((1, HID), lambda p, i: (0, 0)),
            pl.BlockSpec((OUT, HID), lambda p, i: (0, 0)),
            pl.BlockSpec((1, OUT), lambda p, i: (0, 0)),
        ],
        out_specs=pl.BlockSpec((OUT, BL), lambda p, i: (0, i)),
        out_shape=jax.ShapeDtypeStruct((OUT, N_ROWS), jnp.float32),
        scratch_shapes=[
            pltpu.VMEM((IN_CH, IN_CH), jnp.float32),
            pltpu.VMEM((MROW, IN_CH), jnp.float32),
            pltpu.VMEM((IN_CH, HID), jnp.bfloat16),
            pltpu.VMEM((HID, 1), jnp.float32),
            pltpu.VMEM((OUT, 1), jnp.float32),
            pltpu.VMEM((OUT, HID), jnp.bfloat16),
            pltpu.VMEM((IN_CH, NBL * BL), jnp.bfloat16),
        ],
        compiler_params=pltpu.CompilerParams(
            dimension_semantics=("arbitrary", "arbitrary")),
    )(feat.T, W1.T, gamma[None, :], beta[None, :], W2, b2[None, :])

    return outT.T
